# baseline (speedup 1.0000x reference)
import functools

import jax
import jax.numpy as jnp
from jax import lax
from jax.experimental import pallas as pl
from jax.experimental.pallas import tpu as pltpu
from jax.experimental.pallas import tpu_sc as plsc

BATCH = 4096
HIST = 200
EMBED = 32

NC = 2
NS = 16
NW = NC * NS
BG = BATCH // NW

_mesh = plsc.VectorSubcoreMesh(core_axis_name="c", subcore_axis_name="s")


@functools.partial(
    pl.kernel,
    mesh=_mesh,
    out_type=jax.ShapeDtypeStruct((HIST, 4, 32, 1024), jnp.float32),
    compiler_params=pltpu.CompilerParams(use_tc_tiling_on_sc=False,
                                         needs_layout_passes=False),
    scratch_types=[
        pltpu.VMEM((HIST, BG), jnp.int32),
        pltpu.VMEM((BG, EMBED), jnp.float32),
        pltpu.VMEM((BG, EMBED), jnp.float32),
        pltpu.VMEM((4096,), jnp.float32),
        pltpu.VMEM((4096,), jnp.float32),
        pltpu.SemaphoreType.DMA,
        pltpu.SemaphoreType.DMA,
        pltpu.SemaphoreType.DMA,
        pltpu.SemaphoreType.DMA,
    ],
)
def _embed_kernel(table_hbm, idxt_hbm, out_hbm, idx_v, g0, g1, t0, t1,
                  gsem0, gsem1, wsem0, wsem1):
    wid = lax.axis_index("s") * NC + lax.axis_index("c")
    b0 = wid * BG

    pltpu.sync_copy(idxt_hbm.at[:, pl.ds(b0, BG)], idx_v)

    g = (g0, g1)
    t = (t0, t1)
    gsems = (gsem0, gsem1)
    wsems = (wsem0, wsem1)

    def start_gather(h, b):
        pltpu.async_copy(table_hbm.at[idx_v.at[h]], g[b], gsems[b])

    def wait_gather(h, b):
        pltpu.make_async_copy(table_hbm.at[idx_v.at[h]], g[b], gsems[b]).wait()

    def start_write(h, b):
        for dh in range(4):
            pltpu.async_copy(t[b].at[pl.ds(dh * 1024, 1024)],
                             out_hbm.at[h, dh, wid], wsems[b])

    def wait_write(h, b):
        for dh in range(4):
            pltpu.make_async_copy(t[b].at[pl.ds(dh * 1024, 1024)],
                                  out_hbm.at[h, dh, wid], wsems[b]).wait()

    def transpose(b):
        for m in range(8):
            rows = lax.iota(jnp.int32, 16) + (m * 16)
            for d in range(EMBED):
                cols = jnp.full((16,), d, jnp.int32)
                vec = plsc.load_gather(g[b], [rows, cols])
                t[b][pl.ds(d * 128 + m * 16, 16)] = vec

    start_gather(0, 0)
    start_gather(1, 1)

    @pl.loop(0, HIST, step=2)
    def _body(j):
        for b in range(2):
            h = j + b
            wait_gather(h, b)

            @pl.when(h >= 2)
            def _():
                wait_write(h - 2, b)

            transpose(b)
            start_write(h, b)

            @pl.when(h + 2 < HIST)
            def _():
                start_gather(h + 2, b)

    wait_write(HIST - 2, 0)
    wait_write(HIST - 1, 1)


def kernel(input, table):
    idxt = input.astype(jnp.int32).T
    out = _embed_kernel(table, idxt)
    return (out.reshape(HIST, 4, 32, 8, 128)
               .transpose(2, 4, 0, 1, 3)
               .reshape(BATCH, HIST, EMBED))


# speedup vs baseline: 1.7263x; 1.7263x over previous
import functools

import jax
import jax.numpy as jnp
from jax import lax
from jax.experimental import pallas as pl
from jax.experimental.pallas import tpu as pltpu
from jax.experimental.pallas import tpu_sc as plsc

BATCH = 4096
HIST = 200
EMBED = 32

NC = 2
NS = 16
NW = NC * NS
BG = BATCH // NW

_mesh = plsc.VectorSubcoreMesh(core_axis_name="c", subcore_axis_name="s")


@functools.partial(
    pl.kernel,
    mesh=_mesh,
    out_type=jax.ShapeDtypeStruct((HIST, 4, 32, 1024), jnp.float32),
    compiler_params=pltpu.CompilerParams(use_tc_tiling_on_sc=False,
                                         needs_layout_passes=False),
    scratch_types=[
        pltpu.VMEM((HIST, BG), jnp.int32),
        pltpu.VMEM((BG, EMBED), jnp.float32),
        pltpu.VMEM((BG, EMBED), jnp.float32),
        pltpu.VMEM((4096,), jnp.float32),
        pltpu.VMEM((4096,), jnp.float32),
        pltpu.SemaphoreType.DMA,
        pltpu.SemaphoreType.DMA,
        pltpu.SemaphoreType.DMA,
        pltpu.SemaphoreType.DMA,
    ],
)
def _embed_kernel(table_hbm, idxt_hbm, out_hbm, idx_v, g0, g1, t0, t1,
                  gsem0, gsem1, wsem0, wsem1):
    wid = lax.axis_index("s") * NC + lax.axis_index("c")
    b0 = wid * BG

    pltpu.sync_copy(idxt_hbm.at[:, pl.ds(b0, BG)], idx_v)

    g = (g0, g1)
    t = (t0, t1)
    gsems = (gsem0, gsem1)
    wsems = (wsem0, wsem1)

    def start_gather(h, b):
        pltpu.async_copy(table_hbm.at[idx_v.at[h]], g[b], gsems[b])

    def wait_gather(h, b):
        pltpu.make_async_copy(table_hbm.at[idx_v.at[h]], g[b], gsems[b]).wait()

    def start_write(h, b):
        for dh in range(4):
            pltpu.async_copy(t[b].at[pl.ds(dh * 1024, 1024)],
                             out_hbm.at[h, dh, wid], wsems[b])

    def wait_write(h, b):
        for dh in range(4):
            pltpu.make_async_copy(t[b].at[pl.ds(dh * 1024, 1024)],
                                  out_hbm.at[h, dh, wid], wsems[b]).wait()

    ii = lax.iota(jnp.int32, 16)
    rot = [(ii + j) & 15 for j in range(16)]
    dstb = [rot[j] * 128 + ii for j in range(16)]

    def transpose(b):
        @pl.loop(0, BG // 16)
        def _rows(m):
            rows = ii + m * 16
            for d0 in (0, 16):
                for j in range(16):
                    cols = rot[j] + d0
                    vec = plsc.load_gather(g[b], [rows, cols])
                    plsc.store_scatter(t[b], [dstb[j] + (d0 * 128) + m * 16],
                                       vec)

    start_gather(0, 0)
    start_gather(1, 1)

    @pl.loop(0, HIST, step=2)
    def _body(j):
        for b in range(2):
            h = j + b
            wait_gather(h, b)

            @pl.when(h >= 2)
            def _():
                wait_write(h - 2, b)

            transpose(b)
            start_write(h, b)

            @pl.when(h + 2 < HIST)
            def _():
                start_gather(h + 2, b)

    wait_write(HIST - 2, 0)
    wait_write(HIST - 1, 1)


def kernel(input, table):
    idxt = input.astype(jnp.int32).T
    out = _embed_kernel(table, idxt)
    return (out.reshape(HIST, 4, 32, 8, 128)
               .transpose(2, 4, 0, 1, 3)
               .reshape(BATCH, HIST, EMBED))


# speedup vs baseline: 1.7543x; 1.0162x over previous
import functools

import jax
import jax.numpy as jnp
from jax import lax
from jax.experimental import pallas as pl
from jax.experimental.pallas import tpu as pltpu
from jax.experimental.pallas import tpu_sc as plsc

BATCH = 4096
HIST = 200
EMBED = 32

NC = 2
NS = 16
NW = NC * NS
BG = BATCH // NW

_mesh = plsc.VectorSubcoreMesh(core_axis_name="c", subcore_axis_name="s")


TBL_STRIDE = 4


@functools.partial(
    pl.kernel,
    mesh=_mesh,
    out_type=jax.ShapeDtypeStruct((HIST, 4, 32, 1024), jnp.float32),
    compiler_params=pltpu.CompilerParams(use_tc_tiling_on_sc=False,
                                         needs_layout_passes=False),
    scratch_types=[
        pltpu.VMEM((HIST, BG), jnp.int32),
        pltpu.VMEM((BG, EMBED), jnp.float32),
        pltpu.VMEM((BG, EMBED), jnp.float32),
        pltpu.VMEM((4096,), jnp.float32),
        pltpu.VMEM((4096,), jnp.float32),
        pltpu.SemaphoreType.DMA,
        pltpu.SemaphoreType.DMA,
        pltpu.SemaphoreType.DMA,
        pltpu.SemaphoreType.DMA,
    ],
)
def _embed_kernel(table_hbm, idxt_hbm, out_hbm, idx_v, g0, g1, t0, t1,
                  gsem0, gsem1, wsem0, wsem1):
    wid = lax.axis_index("s") * NC + lax.axis_index("c")
    b0 = wid * BG

    pltpu.sync_copy(idxt_hbm.at[:, pl.ds(b0, BG)], idx_v)

    g = (g0, g1)
    t = (t0, t1)
    gsems = (gsem0, gsem1)
    wsems = (wsem0, wsem1)

    def start_gather(h, b):
        pltpu.async_copy(table_hbm.at[idx_v.at[h]], g[b], gsems[b])

    def wait_gather(h, b):
        pltpu.make_async_copy(table_hbm.at[idx_v.at[h]], g[b], gsems[b]).wait()

    def start_write(h, b):
        for dh in range(4):
            pltpu.async_copy(t[b].at[pl.ds(dh * 1024, 1024)],
                             out_hbm.at[h, dh, wid], wsems[b])

    def wait_write(h, b):
        for dh in range(4):
            pltpu.make_async_copy(t[b].at[pl.ds(dh * 1024, 1024)],
                                  out_hbm.at[h, dh, wid], wsems[b]).wait()

    ii = lax.iota(jnp.int32, 16)
    rot = [(ii + j) & 15 for j in range(16)]
    dstb = [rot[j] * 128 + ii for j in range(16)]

    def transpose(b):
        @pl.loop(0, BG // 16)
        def _rows(m):
            rows = ii + m * 16
            for d0 in (0, 16):
                for j in range(16):
                    cols = rot[j] + d0
                    vec = plsc.load_gather(g[b], [rows, cols])
                    plsc.store_scatter(t[b], [dstb[j] + (d0 * 128) + m * 16],
                                       vec)

    start_gather(0, 0)
    start_gather(1, 1)

    @pl.loop(0, HIST, step=2)
    def _body(j):
        for b in range(2):
            h = j + b
            wait_gather(h, b)

            @pl.when(h >= 2)
            def _():
                wait_write(h - 2, b)

            transpose(b)
            start_write(h, b)

            @pl.when(h + 2 < HIST)
            def _():
                start_gather(h + 2, b)

    wait_write(HIST - 2, 0)
    wait_write(HIST - 1, 1)


def kernel(input, table):
    idxt = input.astype(jnp.int32).T * TBL_STRIDE
    tbl = jnp.pad(table, ((0, 0), (0, 96)))
    tbl = tbl.reshape(table.shape[0] * TBL_STRIDE, EMBED)
    out = _embed_kernel(tbl, idxt)
    return (out.reshape(HIST, 4, 32, 8, 128)
               .transpose(2, 4, 0, 1, 3)
               .reshape(BATCH, HIST, EMBED))
